# baseline (speedup 1.0000x reference)
import functools

import jax
import jax.numpy as jnp
from jax import lax
from jax.experimental import pallas as pl
from jax.experimental.pallas import tpu as pltpu
from jax.experimental.pallas import tpu_sc as plsc

B = 4096
L = 50
D = 64
N = B * L
NW = 32
PER_W = N // NW
CHUNK = 128
NCH = PER_W // CHUNK

_mesh = plsc.VectorSubcoreMesh(core_axis_name="c", subcore_axis_name="s")


@functools.partial(
    pl.kernel,
    mesh=_mesh,
    out_type=jax.ShapeDtypeStruct((N, D), jnp.float32),
    scratch_types=[
        pltpu.VMEM((NCH, CHUNK), jnp.int32),
        pltpu.VMEM((CHUNK, D), jnp.float32),
        pltpu.SemaphoreType.DMA,
    ],
    compiler_params=pltpu.CompilerParams(use_tc_tiling_on_sc=False),
)
def _sc_gather(idx_hbm, table_hbm, out_hbm, idx_v, buf, gsem):
    wid = lax.axis_index("s") * 2 + lax.axis_index("c")
    base = wid * PER_W
    pltpu.sync_copy(idx_hbm.at[wid], idx_v)

    def chunk_body(j, carry):
        pltpu.async_copy(table_hbm.at[idx_v.at[j]], buf, gsem).wait()
        pltpu.sync_copy(buf, out_hbm.at[pl.ds(base + j * CHUNK, CHUNK)])
        return carry

    lax.fori_loop(0, NCH, chunk_body, 0)


def kernel(ts, table):
    idx = ts.reshape(NW, NCH, CHUNK)
    out = _sc_gather(idx, table)
    return out.reshape(B, L, D)


# speedup vs baseline: 1.0448x; 1.0448x over previous
import functools

import jax
import jax.numpy as jnp
from jax import lax
from jax.experimental import pallas as pl
from jax.experimental.pallas import tpu as pltpu
from jax.experimental.pallas import tpu_sc as plsc

B = 4096
L = 50
D = 64
N = B * L
NW = 32
PER_W = N // NW
CHUNK = 128
NCH = PER_W // CHUNK
R = 5

_mesh = plsc.VectorSubcoreMesh(core_axis_name="c", subcore_axis_name="s")


@functools.partial(
    pl.kernel,
    mesh=_mesh,
    out_type=jax.ShapeDtypeStruct((N, D), jnp.float32),
    scratch_types=[
        pltpu.VMEM((NCH, CHUNK), jnp.int32),
        pltpu.VMEM((R, CHUNK, D), jnp.float32),
        [pltpu.SemaphoreType.DMA] * R,
        [pltpu.SemaphoreType.DMA] * R,
    ],
    compiler_params=pltpu.CompilerParams(use_tc_tiling_on_sc=False),
)
def _sc_gather(idx_hbm, table_hbm, out_hbm, idx_v, buf, gsems, ssems):
    wid = lax.axis_index("s") * 2 + lax.axis_index("c")
    base = wid * PER_W
    pltpu.sync_copy(idx_hbm.at[wid], idx_v)

    def fire_gather(j, slot):
        pltpu.async_copy(table_hbm.at[idx_v.at[j]], buf.at[slot], gsems[slot])

    def wait_gather(j, slot):
        pltpu.make_async_copy(
            table_hbm.at[idx_v.at[j]], buf.at[slot], gsems[slot]
        ).wait()

    def fire_scatter(j, slot):
        pltpu.async_copy(
            buf.at[slot], out_hbm.at[pl.ds(base + j * CHUNK, CHUNK)], ssems[slot]
        )

    def wait_scatter(j, slot):
        pltpu.make_async_copy(
            buf.at[slot], out_hbm.at[pl.ds(base + j * CHUNK, CHUNK)], ssems[slot]
        ).wait()

    for j in range(R - 1):
        fire_gather(j, j)

    def outer(jo, carry):
        for b in range(R):
            j = jo * R + b
            wait_gather(j, b)
            fire_scatter(j, b)
            prev = (b - 1) % R

            @pl.when(j >= 1)
            def _():
                wait_scatter(j - 1, prev)

            @pl.when(j + R - 1 < NCH)
            def _():
                fire_gather(j + R - 1, prev)

        return carry

    lax.fori_loop(0, NCH // R, outer, 0)
    wait_scatter(NCH - 1, (NCH - 1) % R)


def kernel(ts, table):
    idx = ts.reshape(NW, NCH, CHUNK)
    out = _sc_gather(idx, table)
    return out.reshape(B, L, D)
